# baseline (speedup 1.0000x reference)
import functools

import jax
import jax.numpy as jnp
from jax import lax
from jax.experimental import pallas as pl
from jax.experimental.pallas import tpu as pltpu
from jax.experimental.pallas import tpu_sc as plsc

B = 16384
D = 64
W = 128
NC = 2
NS = 16
NW = NC * NS
CHUNK = B // NW
NIDX = 128
NJ = CHUNK // NIDX
LANES = 16
WAVE = 256
NWAVE = CHUNK // WAVE
JPW = WAVE // NIDX

_SC_KW = dict(
    mesh=plsc.VectorSubcoreMesh(core_axis_name="c", subcore_axis_name="s"),
    compiler_params=pltpu.CompilerParams(
        needs_layout_passes=False, use_tc_tiling_on_sc=False),
)


def _stage_ids(ids_hbm, g_ref, h_ref, base, n_rows):
    pltpu.sync_copy(ids_hbm.at[pl.ds(base, NJ)], g_ref)
    for a in range(NJ):
        for k in range(NIDX // LANES):
            sl = pl.ds(k * LANES, LANES)
            u = g_ref[a, sl]
            r = jnp.where(u == 0, n_rows - 1, u - 1)
            g_ref[a, sl] = jnp.right_shift(r, 1)
            if h_ref is not None:
                h_ref[pl.ds(a * NIDX + k * LANES, LANES)] = \
                    jnp.bitwise_and(r, 1)


@functools.partial(
    pl.kernel,
    out_type=jax.ShapeDtypeStruct((B, W), jnp.float32),
    scratch_types=[
        pltpu.VMEM((NJ, NIDX), jnp.int32),
        pltpu.VMEM((WAVE, W), jnp.float32),
        pltpu.SemaphoreType.DMA,
    ],
    **_SC_KW,
)
def _gather_u(uid_hbm, p_hbm, um_hbm, g_u, rows, sem):
    c = lax.axis_index("c")
    s = lax.axis_index("s")
    wid = s * NC + c
    _stage_ids(uid_hbm, g_u, None, wid * NJ, 2 * p_hbm.shape[0])
    for w in range(NWAVE):
        cps = [pltpu.async_copy(
            p_hbm.at[g_u.at[w * JPW + j]],
            rows.at[pl.ds(j * NIDX, NIDX)], sem) for j in range(JPW)]
        for cp in cps:
            cp.wait()
        pltpu.sync_copy(
            rows, um_hbm.at[pl.ds(wid * CHUNK + w * WAVE, WAVE)])


@functools.partial(
    pl.kernel,
    out_type=jax.ShapeDtypeStruct((B,), jnp.float32),
    scratch_types=[
        pltpu.VMEM((NJ, NIDX), jnp.int32),
        pltpu.VMEM((NJ, NIDX), jnp.int32),
        pltpu.VMEM((CHUNK,), jnp.int32),
        pltpu.VMEM((CHUNK,), jnp.int32),
        pltpu.VMEM((WAVE, W), jnp.float32),
        pltpu.VMEM((WAVE, W), jnp.float32),
        pltpu.VMEM((LANES * LANES,), jnp.float32),
        pltpu.VMEM((CHUNK,), jnp.float32),
        pltpu.SemaphoreType.DMA,
    ],
    **_SC_KW,
)
def _gather_i_dot(uid_hbm, iid_hbm, q_hbm, um_hbm, out_hbm,
                  g_u, g_i, h_u, h_i, rows_u, rows_i, m_v, out_v, sem):
    c = lax.axis_index("c")
    s = lax.axis_index("s")
    wid = s * NC + c
    base = wid * NJ
    _stage_ids(uid_hbm, g_u, h_u, base, 2 * q_hbm.shape[0])
    _stage_ids(iid_hbm, g_i, h_i, base, 2 * q_hbm.shape[0])

    zero16 = jnp.zeros((LANES,), jnp.int32)
    col0 = lax.iota(jnp.int32, LANES) * LANES

    for w in range(NWAVE):
        cps = [pltpu.async_copy(
            q_hbm.at[g_i.at[w * JPW + j]],
            rows_i.at[pl.ds(j * NIDX, NIDX)], sem) for j in range(JPW)]
        cps.append(pltpu.async_copy(
            um_hbm.at[pl.ds(wid * CHUNK + w * WAVE, WAVE)], rows_u, sem))
        for cp in cps:
            cp.wait()

        def group(g16, carry):
            for l in range(LANES):
                r = g16 * LANES + l
                gidx = w * WAVE + r + zero16
                hu = plsc.load_gather(h_u, [gidx])
                hi = plsc.load_gather(h_i, [gidx])
                acc = None
                for jj in range(D // LANES):
                    lo = pl.ds(jj * LANES, LANES)
                    hi_sl = pl.ds(D + jj * LANES, LANES)
                    us = jnp.where(hu == 0, rows_u[r, lo], rows_u[r, hi_sl])
                    vs = jnp.where(hi == 0, rows_i[r, lo], rows_i[r, hi_sl])
                    acc = us * vs if acc is None else acc + us * vs
                m_v[pl.ds(l * LANES, LANES)] = acc
            res = plsc.load_gather(m_v, [col0])
            for l in range(1, LANES):
                res = res + plsc.load_gather(m_v, [col0 + l])
            out_v[pl.ds(w * WAVE + g16 * LANES, LANES)] = res
            return carry

        lax.fori_loop(0, WAVE // LANES, group, 0)

    pltpu.sync_copy(out_v, out_hbm.at[pl.ds(wid * CHUNK, CHUNK)])


def kernel(avg_score, user_id, item_id, p, q):
    del avg_score
    uid2 = user_id.reshape(B // NIDX, NIDX)
    iid2 = item_id.reshape(B // NIDX, NIDX)
    pr = p.reshape(p.shape[0] // 2, W)
    qr = q.reshape(q.shape[0] // 2, W)
    um = _gather_u(uid2, pr)
    out = _gather_i_dot(uid2, iid2, qr, um)
    return out.reshape(B, 1, 1)
